# baseline (speedup 1.0000x reference)
import functools

import jax
import jax.numpy as jnp
from jax import lax
from jax.experimental import pallas as pl
from jax.experimental.pallas import tpu as pltpu
from jax.experimental.pallas import tpu_sc as plsc

_NC = 2
_NS = 16
_NW = _NC * _NS
_NBUF = 4
_KA = 88


@functools.lru_cache(maxsize=None)
def _make_kernel(B, L, D):
    KA = _KA
    KB = L - KA
    rpw = B // _NW
    cpw = 2 * rpw
    assert B % _NW == 0 and cpw % _NBUF == 0 and D % 16 == 0
    assert KA % 8 == 0 and KB % 8 == 0 and KA <= 128 and KB <= 128

    mesh = plsc.VectorSubcoreMesh(core_axis_name="c", subcore_axis_name="s")

    @functools.partial(
        pl.kernel,
        mesh=mesh,
        out_type=jax.ShapeDtypeStruct((B, L, D), jnp.float32),
        scratch_types=[
            pltpu.VMEM((rpw, KA), jnp.int32),
            pltpu.VMEM((rpw, KB), jnp.int32),
            pltpu.VMEM((2, KA, D), jnp.float32),
            pltpu.VMEM((2, KB, D), jnp.float32),
            pltpu.VMEM((L, D), jnp.float32),
            [pltpu.SemaphoreType.DMA] * _NBUF,
            [pltpu.SemaphoreType.DMA] * _NBUF,
        ],
    )
    def k(ia_hbm, ib_hbm, table_hbm, pos_hbm, out_hbm, idx_a, idx_b,
          rows_a, rows_b, pos_v, gsems, osems):
        wid = lax.axis_index("s") * _NC + lax.axis_index("c")
        row0 = wid * rpw

        pltpu.sync_copy(pos_hbm, pos_v)
        pltpu.sync_copy(ia_hbm.at[pl.ds(row0, rpw)], idx_a)
        pltpu.sync_copy(ib_hbm.at[pl.ds(row0, rpw)], idx_b)

        def gather(q, b):
            if b % 2 == 0:
                return pltpu.make_async_copy(
                    table_hbm.at[idx_a.at[q // 2]], rows_a.at[b // 2],
                    gsems[b])
            return pltpu.make_async_copy(
                table_hbm.at[idx_b.at[q // 2]], rows_b.at[b // 2], gsems[b])

        def wback(q, b):
            if b % 2 == 0:
                return pltpu.make_async_copy(
                    rows_a.at[b // 2],
                    out_hbm.at[row0 + q // 2, pl.ds(0, KA)], osems[b])
            return pltpu.make_async_copy(
                rows_b.at[b // 2],
                out_hbm.at[row0 + q // 2, pl.ds(KA, KB)], osems[b])

        for b in range(_NBUF - 1):
            gather(b, b).start()

        def super_body(i, carry):
            g = i * _NBUF
            for b in range(_NBUF):
                q = g + b
                gather(q, b).wait()

                rows_v = rows_a if b % 2 == 0 else rows_b
                n_tok = KA if b % 2 == 0 else KB
                off = 0 if b % 2 == 0 else KA

                @plsc.parallel_loop(0, n_tok, unroll=4)
                def tok_body(t, rows_v=rows_v, b=b, off=off):
                    for d in range(D // 16):
                        sl = pl.ds(d * 16, 16)
                        plsc.addupdate(rows_v.at[b // 2, t, sl],
                                       pos_v[off + t, sl])
                wback(q, b).start()

                pb = (b - 1) % _NBUF

                @pl.when(q >= 1)
                def _(q=q, pb=pb):
                    wback(q - 1, pb).wait()

                @pl.when(q + _NBUF - 1 < cpw)
                def _(q=q, pb=pb):
                    gather(q + _NBUF - 1, pb).start()

            return carry

        lax.fori_loop(0, cpw // _NBUF, super_body, 0)
        wback(cpw - 1, _NBUF - 1).wait()

    return k


def kernel(inputs, token_table, pos_table):
    B, L = inputs.shape
    _, D = token_table.shape
    k = _make_kernel(B, L, D)
    idx = inputs.astype(jnp.int32)
    return k(idx[:, :_KA], idx[:, _KA:], token_table, pos_table)


# speedup vs baseline: 1.0093x; 1.0093x over previous
import functools

import jax
import jax.numpy as jnp
from jax import lax
from jax.experimental import pallas as pl
from jax.experimental.pallas import tpu as pltpu
from jax.experimental.pallas import tpu_sc as plsc

_NC = 2
_NS = 16
_NW = _NC * _NS
_NBUF = 4
_KA = 88


@functools.lru_cache(maxsize=None)
def _make_kernel(B, L, D):
    KA = _KA
    KB = L - KA
    rpw = B // _NW
    cpw = 2 * rpw
    assert B % _NW == 0 and cpw % _NBUF == 0 and D % 16 == 0
    assert KA % 8 == 0 and KB % 8 == 0 and KA <= 128 and KB <= 128

    mesh = plsc.VectorSubcoreMesh(core_axis_name="c", subcore_axis_name="s")

    @functools.partial(
        pl.kernel,
        mesh=mesh,
        out_type=jax.ShapeDtypeStruct((B, L, D), jnp.float32),
        scratch_types=[
            pltpu.VMEM((rpw, KA), jnp.int32),
            pltpu.VMEM((rpw, KB), jnp.int32),
            pltpu.VMEM((2, KA, D), jnp.float32),
            pltpu.VMEM((2, KB, D), jnp.float32),
            pltpu.VMEM((L, D), jnp.float32),
            [pltpu.SemaphoreType.DMA] * _NBUF,
            [pltpu.SemaphoreType.DMA] * _NBUF,
        ],
    )
    def k(ia_hbm, ib_hbm, table_hbm, pos_hbm, out_hbm, idx_a, idx_b,
          rows_a, rows_b, pos_v, gsems, osems):
        wid = lax.axis_index("s") * _NC + lax.axis_index("c")
        row0 = wid * rpw

        pltpu.sync_copy(ia_hbm.at[pl.ds(row0, rpw)], idx_a)
        pltpu.sync_copy(ib_hbm.at[pl.ds(row0, rpw)], idx_b)

        def gather(q, b):
            if b % 2 == 0:
                return pltpu.make_async_copy(
                    table_hbm.at[idx_a.at[q // 2]], rows_a.at[b // 2],
                    gsems[b])
            return pltpu.make_async_copy(
                table_hbm.at[idx_b.at[q // 2]], rows_b.at[b // 2], gsems[b])

        def wback(q, b):
            if b % 2 == 0:
                return pltpu.make_async_copy(
                    rows_a.at[b // 2],
                    out_hbm.at[row0 + q // 2, pl.ds(0, KA)], osems[b])
            return pltpu.make_async_copy(
                rows_b.at[b // 2],
                out_hbm.at[row0 + q // 2, pl.ds(KA, KB)], osems[b])

        for b in range(_NBUF - 1):
            gather(b, b).start()
        pltpu.sync_copy(pos_hbm, pos_v)

        def super_body(i, carry):
            g = i * _NBUF
            for b in range(_NBUF):
                q = g + b
                gather(q, b).wait()

                rows_v = rows_a if b % 2 == 0 else rows_b
                n_tok = KA if b % 2 == 0 else KB
                off = 0 if b % 2 == 0 else KA

                def tok_body(t, c2, rows_v=rows_v, b=b, off=off):
                    for d in range(D // 16):
                        sl = pl.ds(d * 16, 16)
                        plsc.addupdate(rows_v.at[b // 2, t, sl],
                                       pos_v[off + t, sl])
                    return c2

                lax.fori_loop(0, n_tok, tok_body, 0)
                wback(q, b).start()

                pb = (b - 1) % _NBUF

                @pl.when(q >= 1)
                def _(q=q, pb=pb):
                    wback(q - 1, pb).wait()

                @pl.when(q + _NBUF - 1 < cpw)
                def _(q=q, pb=pb):
                    gather(q + _NBUF - 1, pb).start()

            return carry

        lax.fori_loop(0, cpw // _NBUF, super_body, 0)
        wback(cpw - 1, _NBUF - 1).wait()

    return k


def kernel(inputs, token_table, pos_table):
    B, L = inputs.shape
    _, D = token_table.shape
    k = _make_kernel(B, L, D)
    idx = inputs.astype(jnp.int32)
    return k(idx[:, :_KA], idx[:, _KA:], token_table, pos_table)
